# baseline (speedup 1.0000x reference)
import jax
import jax.numpy as jnp
from jax import lax
from jax.experimental import pallas as pl
from jax.experimental.pallas import tpu as pltpu

_B, _S, _D, _H = 4, 2048, 1024, 1024
_MT = 2048
_NT = (_B * _S) // _MT


def _logits_body(emb_ref, w1_ref, b1_ref, w2_ref, b2_ref, out_ref):
    x = jnp.dot(emb_ref[...], w1_ref[...], preferred_element_type=jnp.float32)
    x = jnp.maximum(x + b1_ref[...], 0.0)
    lg = jnp.dot(x, w2_ref[...], preferred_element_type=jnp.float32)
    out_ref[...] = lg[:, 0:1] + b2_ref[...]


def _mask_filter_body(lg_ref, emb_ref, mask_ref, ek_ref, filt_ref,
                      mk_scr):
    i = pl.program_id(0)

    @pl.when(i == 0)
    def _select():
        lg = lg_ref[...]
        ek = jnp.sum(jax.nn.sigmoid(lg), axis=1, keepdims=True)
        ek_ref[...] = ek
        k = jnp.maximum(ek.astype(jnp.int32), 32)

        bits = lax.bitcast_convert_type(lg, jnp.int32)
        key = jnp.where(bits < 0, bits ^ jnp.int32(0x7FFFFFFF), bits)

        nonneg = key >= 0
        cnt_nn = jnp.sum(nonneg.astype(jnp.int32), axis=1, keepdims=True)
        in_pos = k <= cnt_nn
        kk = jnp.where(in_pos, k, k - cnt_nn)
        cls = nonneg == in_pos
        m = key & jnp.int32(0x7FFFFFFF)
        p = jnp.zeros_like(k)
        for b_idx in range(30, -1, -1):
            q = p + jnp.int32(1 << b_idx)
            c = jnp.sum(jnp.where(cls & (m >= q), 1, 0), axis=1, keepdims=True)
            p = jnp.where(c >= kk, q, p)
        thr = jnp.where(in_pos, p, p | jnp.int32(-2147483648))

        gt = key > thr
        c_gt = jnp.sum(gt.astype(jnp.int32), axis=1, keepdims=True)
        r = k - c_gt
        tie = key == thr
        idx = lax.broadcasted_iota(jnp.int32, (_B, _S), 1)
        pi = jnp.zeros_like(k)
        for b_idx in range(11, -1, -1):
            qi = pi + jnp.int32(1 << b_idx)
            ci = jnp.sum(jnp.where(tie & (idx < qi), 1, 0), axis=1,
                         keepdims=True)
            pi = jnp.where(ci < r, qi, pi)
        hard = gt | (tie & (idx <= pi))
        mk = hard.astype(jnp.float32)
        mask_ref[...] = mk
        mk_scr[...] = mk

    mk_row = mk_scr[pl.ds(i, 1), :]
    filt_ref[...] = emb_ref[...] * jnp.swapaxes(mk_row, 0, 1)


def kernel(token_embeddings, W1, b1, W2, b2):
    emb2d = token_embeddings.reshape(_B * _S, _D)
    logits_col = pl.pallas_call(
        _logits_body,
        grid=(_NT,),
        in_specs=[
            pl.BlockSpec((_MT, _D), lambda i: (i, 0)),
            pl.BlockSpec((_D, _H), lambda i: (0, 0)),
            pl.BlockSpec((1, _H), lambda i: (0, 0)),
            pl.BlockSpec((_D, 1), lambda i: (0, 0)),
            pl.BlockSpec((1, 1), lambda i: (0, 0)),
        ],
        out_specs=pl.BlockSpec((_MT, 1), lambda i: (i, 0)),
        out_shape=jax.ShapeDtypeStruct((_B * _S, 1), jnp.float32),
    )(emb2d, W1, b1.reshape(1, _H), W2, b2.reshape(1, 1))
    logits = logits_col.reshape(_B, _S)

    mask, ek, filt = pl.pallas_call(
        _mask_filter_body,
        grid=(_B,),
        in_specs=[
            pl.BlockSpec((_B, _S), lambda i: (0, 0)),
            pl.BlockSpec((_S, _D), lambda i: (i, 0)),
        ],
        out_specs=(
            pl.BlockSpec((_B, _S), lambda i: (0, 0)),
            pl.BlockSpec((_B, 1), lambda i: (0, 0)),
            pl.BlockSpec((_S, _D), lambda i: (i, 0)),
        ),
        out_shape=(
            jax.ShapeDtypeStruct((_B, _S), jnp.float32),
            jax.ShapeDtypeStruct((_B, 1), jnp.float32),
            jax.ShapeDtypeStruct((_B * _S, _D), jnp.float32),
        ),
        scratch_shapes=[
            pltpu.VMEM((_B, _S), jnp.float32),
        ],
        compiler_params=pltpu.CompilerParams(
            dimension_semantics=("arbitrary",),
        ),
    )(logits, emb2d)

    return filt.reshape(_B, _S, _D), mask, ek.reshape(_B)
